# baseline (speedup 1.0000x reference)
import jax
import jax.numpy as jnp
from jax import lax
from jax.experimental import pallas as pl
from jax.experimental.pallas import tpu as pltpu
from jax.experimental.pallas import tpu_sc as plsc

_B, _N, _D = 128, 500, 256
_NC, _NS = 2, 16
_NW = _NC * _NS

_SPLIT = 376
_NS_ROWS = _N - _SPLIT
_RW = 8
_REP = 16
_ACT = 16
_TAIL = _NS_ROWS - (_ACT - 1) * _RW
_BB = 16


def _sc_body(table_hbm, out_hbm, rep_v, sem):
    wid = lax.axis_index("s") * _NC + lax.axis_index("c")
    r0 = _SPLIT + wid * _RW
    o0 = wid * _RW

    @pl.when(wid < _ACT)
    def _stage():
        reads = [
            pltpu.async_copy(table_hbm.at[pl.ds(r0, _RW)], rep_v.at[:, j, :], sem)
            for j in range(_REP)
        ]
        for r in reads:
            r.wait()

    @pl.when(wid < _ACT - 1)
    def _main():
        copies = [
            pltpu.async_copy(
                rep_v, out_hbm.at[pl.ds(o0, _RW), pl.ds(j * _REP, _REP), :], sem
            )
            for j in range(_B // _REP)
        ]
        for c in copies:
            c.wait()

    @pl.when(wid == _ACT - 1)
    def _tail():
        copies = [
            pltpu.async_copy(
                rep_v.at[pl.ds(0, _TAIL)],
                out_hbm.at[pl.ds(o0, _TAIL), pl.ds(j * _REP, _REP), :],
                sem,
            )
            for j in range(_B // _REP)
        ]
        for c in copies:
            c.wait()


def _tc_body(emb_ref, out_ref):
    out_ref[...] = jnp.broadcast_to(emb_ref[...][:, None, :], (_SPLIT, _BB, _D))


@jax.jit
def _bcast(embed_weight):
    table_padded = jnp.pad(embed_weight, ((0, 512 - _N), (0, 0)))
    mesh = plsc.VectorSubcoreMesh(core_axis_name="c", subcore_axis_name="s")
    sc_part = pl.kernel(
        _sc_body,
        mesh=mesh,
        out_type=jax.ShapeDtypeStruct((_NS_ROWS, _B, _D), jnp.float32),
        scratch_types=[
            pltpu.VMEM((_RW, _REP, _D), jnp.float32),
            pltpu.SemaphoreType.DMA,
        ],
    )(table_padded)
    tc_part = pl.pallas_call(
        _tc_body,
        grid=(_B // _BB,),
        in_specs=[pl.BlockSpec((_SPLIT, _D), lambda i: (0, 0))],
        out_specs=pl.BlockSpec((_SPLIT, _BB, _D), lambda i: (0, i, 0)),
        out_shape=jax.ShapeDtypeStruct((_SPLIT, _B, _D), jnp.float32),
    )(embed_weight)
    rows_major = jnp.concatenate([tc_part, sc_part], axis=0)
    return jnp.transpose(rows_major, (1, 0, 2))


def kernel(x, embed_weight):
    del x
    return _bcast(embed_weight)


# speedup vs baseline: 2.2298x; 2.2298x over previous
import jax
import jax.numpy as jnp
from jax import lax
from jax.experimental import pallas as pl
from jax.experimental.pallas import tpu as pltpu
from jax.experimental.pallas import tpu_sc as plsc

_B, _N, _D = 128, 500, 256
_NC, _NS = 2, 16
_S = 375
_T = _N - _S
_GP = 128
_GW = 16
_GR = _GP // _GW
_BB = 16


def _sc_gather_body(table_hbm, idx_hbm, out_hbm, idx_v, rows_v, sem):
    wid = lax.axis_index("s") * _NC + lax.axis_index("c")

    @pl.when(wid < _GW)
    def _():
        base = wid * _GR
        pltpu.sync_copy(idx_hbm.at[pl.ds(base, _GR)], idx_v)
        pltpu.async_copy(table_hbm.at[idx_v], rows_v, sem).wait()
        pltpu.sync_copy(rows_v, out_hbm.at[pl.ds(base, _GR)])


def _tc1_body(emb_ref, out_ref):
    out_ref[...] = jnp.broadcast_to(
        emb_ref[pl.ds(0, _S), :][:, None, :], (_S, _BB, _D)
    )


def _tc2_body(full_ref, g_ref, out_ref):
    del full_ref
    out_ref[...] = jnp.broadcast_to(
        g_ref[pl.ds(0, _T), :][:, None, :], (_T, _BB, _D)
    )


@jax.jit
def _bcast(embed_weight):
    idx = jnp.minimum(_S + jnp.arange(_GP, dtype=jnp.int32), _N - 1)
    mesh = plsc.VectorSubcoreMesh(core_axis_name="c", subcore_axis_name="s")
    gathered = pl.kernel(
        _sc_gather_body,
        mesh=mesh,
        out_type=jax.ShapeDtypeStruct((_GP, _D), jnp.float32),
        scratch_types=[
            pltpu.VMEM((_GR,), jnp.int32),
            pltpu.VMEM((_GR, _D), jnp.float32),
            pltpu.SemaphoreType.DMA,
        ],
    )(embed_weight, idx)

    tc1 = pl.pallas_call(
        _tc1_body,
        grid=(_B // _BB,),
        in_specs=[pl.BlockSpec((_N, _D), lambda j: (0, 0))],
        out_specs=pl.BlockSpec((_S, _BB, _D), lambda j: (0, j, 0)),
        out_shape=jax.ShapeDtypeStruct((_N, _B, _D), jnp.float32),
    )(embed_weight)

    rows_major = pl.pallas_call(
        _tc2_body,
        grid=(_B // _BB,),
        in_specs=[
            pl.BlockSpec(memory_space=pl.ANY),
            pl.BlockSpec((_GP, _D), lambda j: (0, 0)),
        ],
        out_specs=pl.BlockSpec((_T, _BB, _D), lambda j: (_S // _T, j, 0)),
        out_shape=jax.ShapeDtypeStruct((_N, _B, _D), jnp.float32),
        input_output_aliases={0: 0},
    )(tc1, gathered)

    return jnp.transpose(rows_major, (1, 0, 2))


def kernel(x, embed_weight):
    del x
    return _bcast(embed_weight)
